# baseline (speedup 1.0000x reference)
import functools

import jax
import jax.numpy as jnp
from jax import lax
from jax.experimental import pallas as pl
from jax.experimental.pallas import tpu as pltpu
from jax.experimental.pallas import tpu_sc as plsc

_NC, _NS, _L = 2, 16, 16
_NW = _NC * _NS
_RC = 16
_SEG = 32
_NBUF = 5
_AHEAD = 3

_TC_BS = 512


def _sc_body(b_lo, n_b, seq, dim, x_hbm, pos_hbm, out_hbm, pos_buf, *rest):
    bufs = rest[:_NBUF]
    sin = rest[_NBUF:2 * _NBUF]
    sout = rest[2 * _NBUF:3 * _NBUF]
    sp = rest[3 * _NBUF]
    seq_per_w = seq // _NW
    n_segs = seq_per_w // _SEG
    sub = _SEG // _RC
    n_chunks = n_segs * n_b * sub
    wid = lax.axis_index("s") * _NC + lax.axis_index("c")
    seq0 = wid * seq_per_w

    def chunk_row(k, base_b):
        seg, r = divmod(k, n_b * sub)
        b, cc = divmod(r, sub)
        return (base_b + b) * seq + seq0 + seg * _SEG + cc * _RC

    def add_chunk(k, s):
        row_base = (k % sub) * _RC
        buf = bufs[s]

        @plsc.parallel_loop(0, _RC)
        def body(r):
            @plsc.parallel_loop(0, dim, step=_L, unroll=4)
            def cols(c):
                p = pos_buf[row_base + r, pl.ds(c, _L)]
                plsc.addupdate(buf.at[r, pl.ds(c, _L)], p)

    def start_in(k):
        s = k % _NBUF
        if out_d[s] is not None:
            out_d[s].wait()
        in_d[s] = pltpu.async_copy(
            x_hbm.at[pl.ds(chunk_row(k, b_lo), _RC)], bufs[s], sin[s])

    pos_d = pltpu.async_copy(pos_hbm.at[pl.ds(seq0, _SEG)], pos_buf, sp)
    in_d = [None] * _NBUF
    out_d = [None] * _NBUF
    for k in range(min(_AHEAD, n_chunks)):
        start_in(k)
    per_seg = n_b * sub
    for k in range(n_chunks):
        if k + _AHEAD < n_chunks:
            start_in(k + _AHEAD)
        s = k % _NBUF
        if k % per_seg == 0:
            pos_d.wait()
        in_d[s].wait()
        add_chunk(k, s)
        out_d[s] = pltpu.async_copy(
            bufs[s], out_hbm.at[pl.ds(chunk_row(k, 0), _RC)], sout[s])
        if k % per_seg == per_seg - 1 and k + 1 < n_chunks:
            pos_d = pltpu.async_copy(
                pos_hbm.at[pl.ds(seq0 + (k // per_seg + 1) * _SEG, _SEG)],
                pos_buf, sp)
    for s in range(_NBUF):
        if out_d[s] is not None:
            out_d[s].wait()


def _sc_add(x, pos_table, b_lo, n_b, seq, dim):
    call = pl.kernel(
        functools.partial(_sc_body, b_lo, n_b, seq, dim),
        out_type=jax.ShapeDtypeStruct((n_b * seq, dim), x.dtype),
        mesh=plsc.VectorSubcoreMesh(core_axis_name="c", subcore_axis_name="s"),
        scratch_types=(
            [pltpu.VMEM((_SEG, dim), jnp.float32)]
            + [pltpu.VMEM((_RC, dim), jnp.float32)] * _NBUF
            + [pltpu.SemaphoreType.DMA] * (2 * _NBUF + 1)
        ),
        compiler_params=pltpu.CompilerParams(use_tc_tiling_on_sc=True),
    )
    return call(x, pos_table)


def _tc_add_kernel(x_ref, p_ref, o_ref):
    o_ref[...] = x_ref[...] + p_ref[...][None, :, :]


def _tc_add(inputs, pos_table, n_b):
    batch, seq, dim = inputs.shape
    return pl.pallas_call(
        _tc_add_kernel,
        grid=(seq // _TC_BS,),
        in_specs=[
            pl.BlockSpec((n_b, _TC_BS, dim), lambda i: (0, i, 0)),
            pl.BlockSpec((_TC_BS, dim), lambda i: (i, 0)),
        ],
        out_specs=pl.BlockSpec((n_b, _TC_BS, dim), lambda i: (0, i, 0)),
        out_shape=jax.ShapeDtypeStruct((n_b, seq, dim), inputs.dtype),
    )(inputs, pos_table)


def kernel(inputs, pos_table):
    batch, seq, dim = inputs.shape
    n_tc = batch - 1
    x = inputs.reshape(batch * seq, dim)
    sc_part = _sc_add(x, pos_table, n_tc, batch - n_tc, seq, dim)
    tc_part = _tc_add(inputs, pos_table, n_tc)
    return jnp.concatenate(
        [tc_part, sc_part.reshape(batch - n_tc, seq, dim)], axis=0)


# speedup vs baseline: 1.3163x; 1.3163x over previous
import functools

import jax
import jax.numpy as jnp
from jax import lax
from jax.experimental import pallas as pl
from jax.experimental.pallas import tpu as pltpu
from jax.experimental.pallas import tpu_sc as plsc

_NC, _NS, _L = 2, 16, 16
_NW = _NC * _NS
_RC = 16
_SEG = 32
_NBUF = 5
_AHEAD = 3

_TC_BS = 512


def _sc_body(b_lo, n_b, seq, dim, x_hbm, pos_hbm, out_hbm, pos_buf, *rest):
    bufs = rest[:_NBUF]
    sin = rest[_NBUF:2 * _NBUF]
    sout = rest[2 * _NBUF:3 * _NBUF]
    sp = rest[3 * _NBUF]
    seq_per_w = seq // _NW
    n_segs = seq_per_w // _SEG
    sub = _SEG // _RC
    n_chunks = n_segs * n_b * sub
    wid = lax.axis_index("s") * _NC + lax.axis_index("c")
    seq0 = wid * seq_per_w

    def chunk_row(k, base_b):
        seg, r = divmod(k, n_b * sub)
        b, cc = divmod(r, sub)
        return (base_b + b) * seq + seq0 + seg * _SEG + cc * _RC

    def add_chunk(k, s):
        row_base = (k % sub) * _RC
        buf = bufs[s]

        @plsc.parallel_loop(0, _RC)
        def body(r):
            @plsc.parallel_loop(0, dim, step=_L, unroll=4)
            def cols(c):
                p = pos_buf[row_base + r, pl.ds(c, _L)]
                plsc.addupdate(buf.at[r, pl.ds(c, _L)], p)

    def start_in(k):
        s = k % _NBUF
        if out_d[s] is not None:
            out_d[s].wait()
        in_d[s] = pltpu.async_copy(
            x_hbm.at[pl.ds(chunk_row(k, b_lo), _RC)], bufs[s], sin[s])

    pos_d = pltpu.async_copy(pos_hbm.at[pl.ds(seq0, _SEG)], pos_buf, sp)
    in_d = [None] * _NBUF
    out_d = [None] * _NBUF
    for k in range(min(_AHEAD, n_chunks)):
        start_in(k)
    per_seg = n_b * sub
    for k in range(n_chunks):
        if k + _AHEAD < n_chunks:
            start_in(k + _AHEAD)
        s = k % _NBUF
        if k % per_seg == 0:
            pos_d.wait()
        in_d[s].wait()
        add_chunk(k, s)
        out_d[s] = pltpu.async_copy(
            bufs[s], out_hbm.at[pl.ds(chunk_row(k, 0), _RC)], sout[s])
        if k % per_seg == per_seg - 1 and k + 1 < n_chunks:
            pos_d = pltpu.async_copy(
                pos_hbm.at[pl.ds(seq0 + (k // per_seg + 1) * _SEG, _SEG)],
                pos_buf, sp)
    for s in range(_NBUF):
        if out_d[s] is not None:
            out_d[s].wait()


def _sc_add(x, pos_table, b_lo, n_b, seq, dim):
    call = pl.kernel(
        functools.partial(_sc_body, b_lo, n_b, seq, dim),
        out_type=jax.ShapeDtypeStruct((n_b * seq, dim), x.dtype),
        mesh=plsc.VectorSubcoreMesh(core_axis_name="c", subcore_axis_name="s"),
        scratch_types=(
            [pltpu.VMEM((_SEG, dim), jnp.float32)]
            + [pltpu.VMEM((_RC, dim), jnp.float32)] * _NBUF
            + [pltpu.SemaphoreType.DMA] * (2 * _NBUF + 1)
        ),
        compiler_params=pltpu.CompilerParams(use_tc_tiling_on_sc=True),
    )
    return call(x, pos_table)


def _tc_add_kernel(x_ref, p_ref, o_ref):
    o_ref[...] = x_ref[...] + p_ref[...][None, :, :]


def _tc_add(inputs, pos_table, n_b):
    batch, seq, dim = inputs.shape
    return pl.pallas_call(
        _tc_add_kernel,
        grid=(seq // _TC_BS,),
        in_specs=[
            pl.BlockSpec((n_b, _TC_BS, dim), lambda i: (0, i, 0)),
            pl.BlockSpec((_TC_BS, dim), lambda i: (i, 0)),
        ],
        out_specs=pl.BlockSpec((n_b, _TC_BS, dim), lambda i: (0, i, 0)),
        out_shape=jax.ShapeDtypeStruct((batch, seq, dim), inputs.dtype),
    )(inputs, pos_table)


def _tc_place_kernel(full_hbm_ref, sc_ref, o_ref):
    del full_hbm_ref
    o_ref[...] = sc_ref[...]


def _tc_place(full, sc_part, b_lo):
    batch, seq, dim = full.shape
    n_b = batch - b_lo
    return pl.pallas_call(
        _tc_place_kernel,
        grid=(seq // _TC_BS,),
        in_specs=[
            pl.BlockSpec(memory_space=pl.ANY),
            pl.BlockSpec((n_b, _TC_BS, dim), lambda i: (0, i, 0)),
        ],
        out_specs=pl.BlockSpec((n_b, _TC_BS, dim), lambda i: (b_lo // n_b, i, 0)),
        out_shape=jax.ShapeDtypeStruct((batch, seq, dim), full.dtype),
        input_output_aliases={0: 0},
    )(full, sc_part.reshape(n_b, seq, dim))


def kernel(inputs, pos_table):
    batch, seq, dim = inputs.shape
    n_tc = batch - 1
    x = inputs.reshape(batch * seq, dim)
    sc_part = _sc_add(x, pos_table, n_tc, batch - n_tc, seq, dim)
    tc_full = _tc_add(inputs, pos_table, n_tc)
    return _tc_place(tc_full, sc_part, n_tc)
